# baseline (speedup 1.0000x reference)
import jax
import jax.numpy as jnp
from jax.experimental import pallas as pl
from jax.experimental.pallas import tpu as pltpu

_N = 128
_G = 16


def _greedy_kernel(x1_ref, x2_ref, out_ref):
    sub = jax.lax.dot_general(
        x1_ref[...], x2_ref[...],
        (((1,), (1,)), ((), ())),
        preferred_element_type=jnp.float32,
    )

    m3_0 = jnp.reshape(sub, (_G, 8, _N))
    lane = jax.lax.broadcasted_iota(jnp.int32, (1, _N), 1)
    lane_f = lane.astype(jnp.float32)
    sub_iota = jax.lax.broadcasted_iota(jnp.int32, (8, _N), 0)
    neg = jnp.float32(-1e30)
    live_thr = jnp.float32(-1e29)
    big_f = jnp.float32(1e9)
    int_min = jnp.int32(-(2**31 - 1) - 1)
    r_iota = jax.lax.broadcasted_iota(jnp.int32, (_N, _N), 0)
    c_iota = jax.lax.broadcasted_iota(jnp.int32, (_N, _N), 1)
    eye_b = r_iota == c_iota
    eye = eye_b.astype(jnp.float32)
    r_col = jax.lax.broadcasted_iota(jnp.int32, (_N, 1), 0)
    r_col_f = r_col.astype(jnp.float32)
    ones_row = jnp.ones((1, _N), jnp.float32)
    ones_col = jnp.ones((_N, 1), jnp.float32)

    def sortable(x):
        b = jax.lax.bitcast_convert_type(x + 0.0, jnp.int32)
        return b ^ (jax.lax.shift_right_arithmetic(b, 31)
                    & jnp.int32(0x7FFFFFFF))

    def split8(s):
        u = s ^ jnp.int32(-(2**31))
        return [(jax.lax.shift_right_logical(u, sh) & 0xFF)
                .astype(jnp.float32) for sh in (24, 16, 8, 0)]

    def join8(b3, b2, b1, b0):
        h = (b3 * 256 + b2).astype(jnp.int32)
        low = (b1 * 256 + b0).astype(jnp.int32)
        return (h - 32768) * 65536 + low

    def cond(carry):
        return jnp.logical_and(carry[3] < _N, carry[4] < _N)

    def body(carry):
        m3, col_kill, total, cnt, ph = carry
        vv = jnp.reshape(m3, (8, 2, 8, _N))
        a, b = vv[:, 0], vv[:, 1]
        keep = a >= b
        pair = jax.lax.broadcasted_iota(jnp.int32, (8, 8, _N), 0)
        g = jnp.where(keep, pair * 2, pair * 2 + 1)
        v = jnp.where(keep, a, b)
        w = jnp.minimum(a, b)
        for n in (4, 2, 1):
            vvv = jnp.reshape(v, (n, 2, 8, _N))
            gg = jnp.reshape(g, (n, 2, 8, _N))
            ww = jnp.reshape(w, (n, 2, 8, _N))
            a, b = vvv[:, 0], vvv[:, 1]
            ga, gb = gg[:, 0], gg[:, 1]
            wa, wb = ww[:, 0], ww[:, 1]
            keep = a >= b
            w = jnp.maximum(jnp.minimum(a, b), jnp.where(keep, wa, wb))
            v = jnp.where(keep, a, b)
            g = jnp.where(keep, ga, gb)
        v1 = v[0]
        w1 = w[0]
        row2_f = (g[0] * 8 + sub_iota).astype(jnp.float32)
        cm = jnp.max(v1, axis=0, keepdims=True)
        rmin_f = jnp.min(jnp.where(v1 == cm, row2_f, big_f),
                         axis=0, keepdims=True)
        win_cell = jnp.logical_and(v1 == cm, row2_f == rmin_f)
        v2 = jnp.max(jnp.where(win_cell, w1, v1), axis=0, keepdims=True)
        cmm = cm + col_kill
        srt = sortable(cmm)
        srt2 = sortable(v2)
        comps = split8(srt) + split8(srt2) + [rmin_f]
        packed = jnp.concatenate(comps, axis=0)
        packed_t = jax.lax.dot_general(
            eye, packed, (((1,), (1,)), ((), ())),
            preferred_element_type=jnp.float32)
        s_t = join8(packed_t[:, 0:1], packed_t[:, 1:2],
                    packed_t[:, 2:3], packed_t[:, 3:4])
        s2_t = join8(packed_t[:, 4:5], packed_t[:, 5:6],
                     packed_t[:, 6:7], packed_t[:, 7:8])
        r_t = packed_t[:, 8:9]
        key_lt = jnp.logical_or(
            r_t < rmin_f,
            jnp.logical_and(r_t == rmin_f, r_col_f < lane_f))
        beats = jnp.logical_or(
            s_t > srt, jnp.logical_and(s_t == srt, key_lt))
        same_row = r_t == rmin_f
        dupmat = jnp.logical_and(beats, same_row)
        dup = jnp.max(dupmat.astype(jnp.float32), axis=0, keepdims=True)
        dup_b = dup >= 0.5
        bound_ge = (s2_t >= srt).astype(jnp.float32)
        cnt_bad = jax.lax.dot_general(
            dup, bound_ge, (((1,), (0,)), ((), ())),
            preferred_element_type=jnp.float32)
        beats_eq = jnp.logical_or(beats, eye_b).astype(jnp.float32)
        cnt_pre = jax.lax.dot_general(
            dup, beats_eq, (((1,), (0,)), ((), ())),
            preferred_element_type=jnp.float32)
        acc_b = jnp.logical_and(
            jnp.logical_and(
                jnp.logical_or(cnt_bad < 0.5, cnt_pre < 0.5),
                jnp.logical_not(dup_b)),
            cmm > live_thr)
        acc = acc_b.astype(jnp.float32)
        rm_mat = (r_col_f == rmin_f).astype(jnp.float32)
        rowkill = jax.lax.dot_general(
            rm_mat, acc, (((1,), (1,)), ((), ())),
            preferred_element_type=jnp.float32)
        cnt_dot = jax.lax.dot_general(
            acc, ones_col, (((1,), (0,)), ((), ())),
            preferred_element_type=jnp.float32)
        m3 = jnp.where(jnp.reshape(rowkill, (_G, 8, 1)) >= 0.5, neg, m3)
        total = total + jnp.where(acc_b, cmm, 0.0)
        col_kill = jnp.where(acc_b, neg, col_kill)
        cnt = cnt + cnt_dot[0, 0].astype(jnp.int32)
        return (m3, col_kill, total, cnt, ph + 1)

    init = (m3_0, jnp.zeros((1, _N), jnp.float32),
            jnp.zeros((1, _N), jnp.float32), jnp.int32(0), jnp.int32(0))
    _, _, total, _, _ = jax.lax.while_loop(cond, body, init)
    out_ref[0, 0] = jnp.tanh(jnp.sum(total) / jnp.float32(_N))


def kernel(x1, x2):
    out = pl.pallas_call(
        _greedy_kernel,
        out_shape=jax.ShapeDtypeStruct((1, 1), jnp.float32),
        out_specs=pl.BlockSpec(memory_space=pltpu.SMEM),
    )(x1, x2)
    return jnp.reshape(out, (1,))


# speedup vs baseline: 1.1726x; 1.1726x over previous
import jax
import jax.numpy as jnp
from jax.experimental import pallas as pl
from jax.experimental.pallas import tpu as pltpu

_N = 128
_G = 16


def _greedy_kernel(x1_ref, x2_ref, out_ref):
    sub = jax.lax.dot_general(
        x1_ref[...], x2_ref[...],
        (((1,), (1,)), ((), ())),
        preferred_element_type=jnp.float32,
    )

    m3_0 = jnp.reshape(sub, (_G, 8, _N))
    lane = jax.lax.broadcasted_iota(jnp.int32, (1, _N), 1)
    lane_f = lane.astype(jnp.float32)
    sub_iota = jax.lax.broadcasted_iota(jnp.int32, (8, _N), 0)
    neg = jnp.float32(-1e30)
    live_thr = jnp.float32(-1e29)
    big_f = jnp.float32(1e9)
    int_min = jnp.int32(-(2**31 - 1) - 1)
    r_iota = jax.lax.broadcasted_iota(jnp.int32, (_N, _N), 0)
    c_iota = jax.lax.broadcasted_iota(jnp.int32, (_N, _N), 1)
    eye_b = r_iota == c_iota
    eye = eye_b.astype(jnp.float32)
    r_col = jax.lax.broadcasted_iota(jnp.int32, (_N, 1), 0)
    r_col_f = r_col.astype(jnp.float32)
    ones_row = jnp.ones((1, _N), jnp.float32)
    ones_col = jnp.ones((_N, 1), jnp.float32)

    def sortable(x):
        b = jax.lax.bitcast_convert_type(x + 0.0, jnp.int32)
        return b ^ (jax.lax.shift_right_arithmetic(b, 31)
                    & jnp.int32(0x7FFFFFFF))

    def split8(s):
        u = s ^ jnp.int32(-(2**31))
        return [(jax.lax.shift_right_logical(u, sh) & 0xFF)
                .astype(jnp.float32) for sh in (24, 16, 8, 0)]

    def join8(b3, b2, b1, b0):
        h = (b3 * 256 + b2).astype(jnp.int32)
        low = (b1 * 256 + b0).astype(jnp.int32)
        return (h - 32768) * 65536 + low

    def cond(carry):
        return jnp.logical_and(carry[3] < _N, carry[4] < _N)

    def body(carry):
        m3, col_kill, total, cnt, ph = carry
        vv = jnp.reshape(m3, (8, 2, 8, _N))
        a, b = vv[:, 0], vv[:, 1]
        keep = a >= b
        pair = jax.lax.broadcasted_iota(jnp.int32, (8, 8, _N), 0)
        g = jnp.where(keep, pair * 2, pair * 2 + 1)
        v = jnp.where(keep, a, b)
        w = jnp.minimum(a, b)
        for n in (4, 2, 1):
            vvv = jnp.reshape(v, (n, 2, 8, _N))
            gg = jnp.reshape(g, (n, 2, 8, _N))
            ww = jnp.reshape(w, (n, 2, 8, _N))
            a, b = vvv[:, 0], vvv[:, 1]
            ga, gb = gg[:, 0], gg[:, 1]
            wa, wb = ww[:, 0], ww[:, 1]
            keep = a >= b
            w = jnp.maximum(jnp.minimum(a, b), jnp.where(keep, wa, wb))
            v = jnp.where(keep, a, b)
            g = jnp.where(keep, ga, gb)
        v1 = v[0]
        w1 = w[0]
        row2_f = (g[0] * 8 + sub_iota).astype(jnp.float32)
        cm = jnp.max(v1, axis=0, keepdims=True)
        rmin_f = jnp.min(jnp.where(v1 == cm, row2_f, big_f),
                         axis=0, keepdims=True)
        win_cell = jnp.logical_and(v1 == cm, row2_f == rmin_f)
        v2 = jnp.max(jnp.where(win_cell, w1, v1), axis=0, keepdims=True)
        cmm = cm + col_kill
        srt = sortable(cmm)
        srt2 = sortable(v2)
        comps = split8(srt) + split8(srt2) + [rmin_f]
        packed = jnp.concatenate(comps, axis=0)
        packed_t = jax.lax.dot_general(
            eye, packed, (((1,), (1,)), ((), ())),
            preferred_element_type=jnp.float32)
        s_t = join8(packed_t[:, 0:1], packed_t[:, 1:2],
                    packed_t[:, 2:3], packed_t[:, 3:4])
        s2_t = join8(packed_t[:, 4:5], packed_t[:, 5:6],
                     packed_t[:, 6:7], packed_t[:, 7:8])
        r_t = packed_t[:, 8:9]
        key_lt = jnp.logical_or(
            r_t < rmin_f,
            jnp.logical_and(r_t == rmin_f, r_col_f < lane_f))
        beats = jnp.logical_or(
            s_t > srt, jnp.logical_and(s_t == srt, key_lt))
        same_row = r_t == rmin_f
        dupmat = jnp.logical_and(beats, same_row)
        dup = jnp.max(dupmat.astype(jnp.float32), axis=0, keepdims=True)
        dup_b = dup >= 0.5
        dupmat2 = jnp.logical_and(
            jnp.logical_not(jnp.logical_or(beats, eye_b)), same_row)
        dup_t = jax.lax.dot_general(
            dupmat2.astype(jnp.float32), ones_col, (((1,), (0,)), ((), ())),
            preferred_element_type=jnp.float32)
        thr = jnp.max(
            jnp.where(jnp.logical_and(dup_t >= 0.5, beats), s2_t, int_min),
            axis=0, keepdims=True)
        uncertain = jnp.logical_and(jnp.logical_not(dup_b), srt <= thr)
        stopped = jax.lax.dot_general(
            uncertain.astype(jnp.float32),
            jnp.logical_or(beats, eye_b).astype(jnp.float32),
            (((1,), (0,)), ((), ())),
            preferred_element_type=jnp.float32)
        acc_b = jnp.logical_and(
            jnp.logical_and(stopped < 0.5, jnp.logical_not(dup_b)),
            cmm > live_thr)
        acc = acc_b.astype(jnp.float32)
        rm_mat = (r_col_f == rmin_f).astype(jnp.float32)
        rowkill = jax.lax.dot_general(
            rm_mat, acc, (((1,), (1,)), ((), ())),
            preferred_element_type=jnp.float32)
        cnt_dot = jax.lax.dot_general(
            acc, ones_col, (((1,), (0,)), ((), ())),
            preferred_element_type=jnp.float32)
        m3 = jnp.where(jnp.reshape(rowkill, (_G, 8, 1)) >= 0.5, neg, m3)
        total = total + jnp.where(acc_b, cmm, 0.0)
        col_kill = jnp.where(acc_b, neg, col_kill)
        cnt = cnt + cnt_dot[0, 0].astype(jnp.int32)
        return (m3, col_kill, total, cnt, ph + 1)

    init = (m3_0, jnp.zeros((1, _N), jnp.float32),
            jnp.zeros((1, _N), jnp.float32), jnp.int32(0), jnp.int32(0))
    _, _, total, _, _ = jax.lax.while_loop(cond, body, init)
    out_ref[0, 0] = jnp.tanh(jnp.sum(total) / jnp.float32(_N))


def kernel(x1, x2):
    out = pl.pallas_call(
        _greedy_kernel,
        out_shape=jax.ShapeDtypeStruct((1, 1), jnp.float32),
        out_specs=pl.BlockSpec(memory_space=pltpu.SMEM),
    )(x1, x2)
    return jnp.reshape(out, (1,))
